# Initial kernel scaffold; baseline (speedup 1.0000x reference)
import jax
import jax.numpy as jnp
from jax import lax
from jax.experimental import pallas as pl
from jax.experimental.pallas import tpu as pltpu
from jax.experimental.pallas import tpu_sc as plsc

_N = 32
_B = 16
_NP = 64
_ROW = 128
_SENTINEL = -(2 ** 20)


def _snake_body(px_hbm, py_hbm, pz_hbm, pv_hbm, out_hbm,
                pxv, pyv, pzv, pvv, outv):
    b = lax.axis_index("s")
    half = lax.axis_index("c")

    pltpu.sync_copy(px_hbm.at[b], pxv)
    pltpu.sync_copy(py_hbm.at[b], pyv)
    pltpu.sync_copy(pz_hbm.at[b], pzv)
    pltpu.sync_copy(pv_hbm.at[b], pvv)

    q0 = half * 16
    cx = pxv[pl.ds(q0, 16)] - 2
    cy = pyv[pl.ds(q0, 16)] - 2
    cz = pzv[pl.ds(q0, 16)] - 2

    zeros = jnp.zeros((16,), jnp.float32)

    def zero_body(t, carry):
        outv[pl.ds(t * 16, 16)] = zeros
        return carry

    lax.fori_loop(0, _ROW, zero_body, 0)

    lane = lax.broadcasted_iota(jnp.int32, (16,), 0)
    base = lane * _ROW

    def scatter_body(j, carry):
        dx = jnp.full((16,), pxv[j], jnp.int32) - cx
        dy = jnp.full((16,), pyv[j], jnp.int32) - cy
        dz = jnp.full((16,), pzv[j], jnp.int32) - cz
        ok = ((dx >= 0) & (dx <= 4) & (dy >= 0) & (dy <= 4)
              & (dz >= 0) & (dz <= 4))
        lin = (dx * 5 + dy) * 5 + dz
        lin = jnp.where(ok, lin, 0)
        val = jnp.full((16,), pvv[j], jnp.float32)
        plsc.store_scatter(outv, [base + lin], val, mask=ok)
        return carry

    lax.fori_loop(0, 63, scatter_body, 0)

    pltpu.sync_copy(outv, out_hbm.at[pl.ds((b * _N + q0) * _ROW, 16 * _ROW)])


@jax.jit
def kernel(acids, mask, idx):
    idx = idx.astype(jnp.int32)
    idx_t = 2 * (idx + (_N - 1))
    mid = (idx_t[:, :-1, :] + idx_t[:, 1:, :]) // 2
    inter_vals = acids[:, :-1] + acids[:, 1:] + 1.0
    inter_mask = mask[:, 1:]

    coords = jnp.concatenate([idx_t, mid], axis=1)
    vals = jnp.concatenate([acids, inter_vals], axis=1)
    valid = jnp.concatenate([mask, inter_mask], axis=1)
    coords = jnp.where(valid[:, :, None], coords, _SENTINEL)

    pad = jnp.full((_B, _NP - coords.shape[1], 3), _SENTINEL, jnp.int32)
    coords = jnp.concatenate([coords, pad], axis=1)
    vals = jnp.concatenate(
        [vals, jnp.zeros((_B, _NP - vals.shape[1]), jnp.float32)], axis=1)

    px = jnp.ascontiguousarray(coords[:, :, 0])
    py = jnp.ascontiguousarray(coords[:, :, 1])
    pz = jnp.ascontiguousarray(coords[:, :, 2])

    snake = pl.kernel(
        _snake_body,
        out_type=jax.ShapeDtypeStruct((_B * _N * _ROW,), jnp.float32),
        mesh=plsc.VectorSubcoreMesh(core_axis_name="c", subcore_axis_name="s"),
        scratch_types=[
            pltpu.VMEM((_NP,), jnp.int32),
            pltpu.VMEM((_NP,), jnp.int32),
            pltpu.VMEM((_NP,), jnp.int32),
            pltpu.VMEM((_NP,), jnp.float32),
            pltpu.VMEM((16 * _ROW,), jnp.float32),
        ],
    )
    flat = snake(px, py, pz, vals)
    out = flat.reshape(_B, _N, _ROW)[:, :, :125].reshape(_B, _N, 5, 5, 5)
    out = out * mask.astype(jnp.float32)[:, :, None, None, None]
    return jnp.expand_dims(out, -1)


# speedup vs baseline: 69.6678x; 69.6678x over previous
import jax
import jax.numpy as jnp
from jax import lax
from jax.experimental import pallas as pl
from jax.experimental.pallas import tpu as pltpu
from jax.experimental.pallas import tpu_sc as plsc

_N = 32
_B = 16
_NP = 64
_ROW = 128
_SENTINEL = -(2 ** 20)


def _snake_body(px_hbm, py_hbm, pz_hbm, pv_hbm, out_hbm,
                pxv, pyv, pzv, pvv, outv):
    b = lax.axis_index("s")
    half = lax.axis_index("c")

    pltpu.sync_copy(px_hbm.at[b], pxv)
    pltpu.sync_copy(py_hbm.at[b], pyv)
    pltpu.sync_copy(pz_hbm.at[b], pzv)
    pltpu.sync_copy(pv_hbm.at[b], pvv)

    q0 = half * 16
    cx = pxv[pl.ds(q0, 16)] - 2
    cy = pyv[pl.ds(q0, 16)] - 2
    cz = pzv[pl.ds(q0, 16)] - 2

    zeros = jnp.zeros((16,), jnp.float32)
    for r in range(16):
        for c8 in range(_ROW // 16):
            outv[r, pl.ds(c8 * 16, 16)] = zeros

    lane = lax.broadcasted_iota(jnp.int32, (16,), 0)
    base = lane * _ROW

    pxg = [pxv[pl.ds(g * 16, 16)] for g in range(4)]
    pyg = [pyv[pl.ds(g * 16, 16)] for g in range(4)]
    pzg = [pzv[pl.ds(g * 16, 16)] for g in range(4)]
    pvg = [pvv[pl.ds(g * 16, 16)] for g in range(4)]

    for j in range(63):
        g, l = divmod(j, 16)
        dx = jnp.full((16,), pxg[g][l], jnp.int32) - cx
        dy = jnp.full((16,), pyg[g][l], jnp.int32) - cy
        dz = jnp.full((16,), pzg[g][l], jnp.int32) - cz
        ok = ((dx >= 0) & (dx <= 4) & (dy >= 0) & (dy <= 4)
              & (dz >= 0) & (dz <= 4))
        lin = (dx * 5 + dy) * 5 + dz
        lin = jnp.where(ok, lin, 0)
        val = jnp.full((16,), pvg[g][l], jnp.float32)
        plsc.store_scatter(outv, [lane, lin], val, mask=ok)

    pltpu.sync_copy(outv, out_hbm.at[pl.ds(b * _N + q0, 16)])


@jax.jit
def kernel(acids, mask, idx):
    idx = idx.astype(jnp.int32)
    idx_t = 2 * (idx + (_N - 1))
    mid = (idx_t[:, :-1, :] + idx_t[:, 1:, :]) // 2
    inter_vals = acids[:, :-1] + acids[:, 1:] + 1.0
    inter_mask = mask[:, 1:]

    coords = jnp.concatenate([idx_t, mid], axis=1)
    vals = jnp.concatenate([acids, inter_vals], axis=1)
    valid = jnp.concatenate([mask, inter_mask], axis=1)
    coords = jnp.where(valid[:, :, None], coords, _SENTINEL)

    pad = jnp.full((_B, _NP - coords.shape[1], 3), _SENTINEL, jnp.int32)
    coords = jnp.concatenate([coords, pad], axis=1)
    vals = jnp.concatenate(
        [vals, jnp.zeros((_B, _NP - vals.shape[1]), jnp.float32)], axis=1)

    px = coords[:, :, 0]
    py = coords[:, :, 1]
    pz = coords[:, :, 2]

    snake = pl.kernel(
        _snake_body,
        out_type=jax.ShapeDtypeStruct((_B * _N, _ROW), jnp.float32),
        mesh=plsc.VectorSubcoreMesh(core_axis_name="c", subcore_axis_name="s"),
        compiler_params=pltpu.CompilerParams(needs_layout_passes=False),
        scratch_types=[
            pltpu.VMEM((_NP,), jnp.int32),
            pltpu.VMEM((_NP,), jnp.int32),
            pltpu.VMEM((_NP,), jnp.int32),
            pltpu.VMEM((_NP,), jnp.float32),
            pltpu.VMEM((16, _ROW), jnp.float32),
        ],
    )
    rows = snake(px, py, pz, vals)
    out = rows.reshape(_B, _N, _ROW)[:, :, :125].reshape(_B, _N, 5, 5, 5)
    out = out * mask.astype(jnp.float32)[:, :, None, None, None]
    return jnp.expand_dims(out, -1)


# speedup vs baseline: 73.8123x; 1.0595x over previous
import jax
import jax.numpy as jnp
from jax import lax
from jax.experimental import pallas as pl
from jax.experimental.pallas import tpu as pltpu
from jax.experimental.pallas import tpu_sc as plsc

_N = 32
_B = 16
_NP = 64
_ROW = 128
_SENTINEL = -(2 ** 20)


def _snake_body(pk_hbm, out_hbm, pkv, outv):
    b = lax.axis_index("s")
    half = lax.axis_index("c")

    pltpu.sync_copy(pk_hbm.at[b], pkv)

    pxg = [pkv[pl.ds(g * 16, 16)] for g in range(4)]
    pyg = [pkv[pl.ds(64 + g * 16, 16)] for g in range(4)]
    pzg = [pkv[pl.ds(128 + g * 16, 16)] for g in range(4)]
    pvg = [plsc.bitcast(pkv[pl.ds(192 + g * 16, 16)], jnp.float32)
           for g in range(4)]

    q0 = half * 16
    cx = pkv[pl.ds(q0, 16)] - 2
    cy = pkv[pl.ds(64 + q0, 16)] - 2
    cz = pkv[pl.ds(128 + q0, 16)] - 2

    zeros = jnp.zeros((16,), jnp.float32)
    for r in range(16):
        for c8 in range(_ROW // 16):
            outv[r, pl.ds(c8 * 16, 16)] = zeros

    lane = lax.broadcasted_iota(jnp.int32, (16,), 0)
    bound = jnp.full((16,), 4, jnp.uint32)

    for j in range(63):
        g, l = divmod(j, 16)
        dx = jnp.full((16,), pxg[g][l], jnp.int32) - cx
        dy = jnp.full((16,), pyg[g][l], jnp.int32) - cy
        dz = jnp.full((16,), pzg[g][l], jnp.int32) - cz
        ok = ((plsc.bitcast(dx, jnp.uint32) <= bound)
              & (plsc.bitcast(dy, jnp.uint32) <= bound)
              & (plsc.bitcast(dz, jnp.uint32) <= bound))
        lin = (dx * 5 + dy) * 5 + dz
        val = jnp.full((16,), pvg[g][l], jnp.float32)
        plsc.store_scatter(outv, [lane, lin], val, mask=ok)

    pltpu.sync_copy(outv, out_hbm.at[pl.ds(b * _N + q0, 16)])


@jax.jit
def kernel(acids, mask, idx):
    idx = idx.astype(jnp.int32)
    idx_t = 2 * (idx + (_N - 1))
    mid = (idx_t[:, :-1, :] + idx_t[:, 1:, :]) // 2
    inter_vals = acids[:, :-1] + acids[:, 1:] + 1.0
    inter_mask = mask[:, 1:]

    coords = jnp.concatenate([idx_t, mid], axis=1)
    vals = jnp.concatenate([acids, inter_vals], axis=1)
    valid = jnp.concatenate([mask, inter_mask], axis=1)
    coords = jnp.where(valid[:, :, None], coords, _SENTINEL)

    zpad = jnp.full((_B, 1), _SENTINEL, jnp.int32)
    packed = jnp.concatenate(
        [coords[:, :, 0], zpad,
         coords[:, :, 1], zpad,
         coords[:, :, 2], zpad,
         lax.bitcast_convert_type(vals, jnp.int32),
         jnp.zeros((_B, 1), jnp.int32)],
        axis=1)

    snake = pl.kernel(
        _snake_body,
        out_type=jax.ShapeDtypeStruct((_B * _N, _ROW), jnp.float32),
        mesh=plsc.VectorSubcoreMesh(core_axis_name="c", subcore_axis_name="s"),
        compiler_params=pltpu.CompilerParams(needs_layout_passes=False),
        scratch_types=[
            pltpu.VMEM((4 * _NP,), jnp.int32),
            pltpu.VMEM((16, _ROW), jnp.float32),
        ],
    )
    rows = snake(packed)
    out = rows.reshape(_B, _N, _ROW)[:, :, :125].reshape(_B, _N, 5, 5, 5)
    out = out * mask.astype(jnp.float32)[:, :, None, None, None]
    return jnp.expand_dims(out, -1)
